# Initial kernel scaffold; baseline (speedup 1.0000x reference)
import jax
import jax.numpy as jnp
from jax.experimental import pallas as pl


def kernel(x, edge_index, edge_attr, We1, be1, We2, be2, Wn1, bn1, Wn2, bn2):
    raise NotImplementedError("write your pallas kernel here")


# speedup vs baseline: 3.1618x; 3.1618x over previous
import functools

import jax
import jax.numpy as jnp
from jax import lax
from jax.experimental import pallas as pl
from jax.experimental.pallas import tpu as pltpu
from jax.experimental.pallas import tpu_sc as plsc

N = 10000
NPAD = 10240
E = 320000
D = 128
DE = 16
H = 128

NC = 2
NS = 16
NW = NC * NS
EPW = E // NW
B = 80
NBATCH = EPW // B
RPS = NPAD // NS


def _pre_body(x_ref, wd_ref, ws_ref, xd_ref, xs_ref):
    xv = x_ref[...]
    dn = (((1,), (0,)), ((), ()))
    xd_ref[...] = lax.dot_general(xv, wd_ref[...], dn,
                                  preferred_element_type=jnp.float32)
    xs_ref[...] = lax.dot_general(xv, ws_ref[...], dn,
                                  preferred_element_type=jnp.float32)


def _ea_body(ea_ref, wa_ref, b1_ref, out_ref):
    dn = (((1,), (0,)), ((), ()))
    out_ref[...] = lax.dot_general(ea_ref[...], wa_ref[...], dn,
                                   preferred_element_type=jnp.float32) + b1_ref[...]


def _node_body(hagg_ref, cts_ref, x_ref, w2_ref, b2_ref, wn1x_ref, wn1a_ref,
               bn1_ref, wn2_ref, bn2_ref, out_ref):
    dn = (((1,), (0,)), ((), ()))
    hsum = hagg_ref[0] + hagg_ref[1]
    aggp = lax.dot_general(hsum, w2_ref[...], dn,
                           preferred_element_type=jnp.float32)
    c = jnp.sum(cts_ref[...], axis=0)
    aggp = aggp + c[:, None] * b2_ref[...][None, :]
    agg = aggp[:N]
    pre = (lax.dot_general(x_ref[...], wn1x_ref[...], dn,
                           preferred_element_type=jnp.float32)
           + lax.dot_general(agg, wn1a_ref[...], dn,
                             preferred_element_type=jnp.float32)
           + bn1_ref[...])
    hn = jnp.maximum(pre, 0.0)
    out_ref[...] = lax.dot_general(hn, wn2_ref[...], dn,
                                   preferred_element_type=jnp.float32) + bn2_ref[...]


def _sc_edge_body(xd_hbm, xs_hbm, ea_hbm, dst_hbm, src_hbm,
                  out_hbm, cnt_hbm,
                  dsti, srci, xdv, xsv, hv, cntv, acc, sem):
    cid = lax.axis_index("c")
    sid = lax.axis_index("s")
    wid = cid * NS + sid

    zero16 = jnp.zeros((16,), jnp.float32)

    def zrow(r, c):
        for j in range(H // 16):
            hv[r, pl.ds(j * 16, 16)] = zero16
        return c
    lax.fori_loop(0, B, zrow, 0)
    for k in range(RPS // B):
        off = pl.multiple_of(sid * RPS + k * B, 8)
        pltpu.sync_copy(hv, acc.at[pl.ds(off, B)])

    def crow(r, c):
        cntv[pl.ds(r * 16, 16)] = zero16
        return c
    lax.fori_loop(0, NPAD // 16, crow, 0)

    plsc.subcore_barrier()

    ebase = wid * EPW
    one16 = jnp.full((16,), 1.0, jnp.float32)

    def batch(b, c):
        base = ebase + b * B
        pltpu.sync_copy(dst_hbm.at[pl.ds(base, B)], dsti)
        pltpu.sync_copy(src_hbm.at[pl.ds(base, B)], srci)
        pltpu.async_copy(xd_hbm.at[dsti], xdv, sem).wait()
        pltpu.async_copy(xs_hbm.at[srci], xsv, sem).wait()
        pltpu.sync_copy(ea_hbm.at[pl.ds(base, B)], hv)

        def row(r, cc):
            for j in range(H // 16):
                s = pl.ds(j * 16, 16)
                t = hv[r, s] + xdv[r, s] + xsv[r, s]
                hv[r, s] = jnp.maximum(t, 0.0)
            return cc
        lax.fori_loop(0, B, row, 0)
        pltpu.sync_copy(hv, acc.at[dsti], add=True)

        for kk in range(B // 16):
            idxv = dsti[pl.ds(kk * 16, 16)]
            plsc.addupdate_scatter(cntv, [idxv], one16)
        return c
    lax.fori_loop(0, NBATCH, batch, 0)

    pltpu.sync_copy(cntv, cnt_hbm.at[wid])

    plsc.subcore_barrier()
    off = pl.multiple_of(sid * RPS, 8)
    pltpu.sync_copy(acc.at[pl.ds(off, RPS)],
                    out_hbm.at[cid, pl.ds(off, RPS)])


_sc_edge = functools.partial(
    pl.kernel,
    mesh=plsc.VectorSubcoreMesh(core_axis_name="c", subcore_axis_name="s"),
    compiler_params=pltpu.CompilerParams(needs_layout_passes=False),
    out_type=(jax.ShapeDtypeStruct((NC, NPAD, H), jnp.float32),
              jax.ShapeDtypeStruct((NW, NPAD), jnp.float32)),
    scratch_types=[
        pltpu.VMEM((B,), jnp.int32),
        pltpu.VMEM((B,), jnp.int32),
        pltpu.VMEM((B, H), jnp.float32),
        pltpu.VMEM((B, H), jnp.float32),
        pltpu.VMEM((B, H), jnp.float32),
        pltpu.VMEM((NPAD,), jnp.float32),
        pltpu.VMEM_SHARED((NPAD, H), jnp.float32),
        pltpu.SemaphoreType.DMA,
    ],
)(_sc_edge_body)


EBLK = 8000


def kernel(x, edge_index, edge_attr, We1, be1, We2, be2, Wn1, bn1, Wn2, bn2):
    dst = edge_index[1].astype(jnp.int32)
    src = edge_index[0].astype(jnp.int32)
    Wa = We1[:DE]
    Wd = We1[DE:DE + D]
    Ws = We1[DE + D:]

    xd, xs = pl.pallas_call(
        _pre_body,
        out_shape=(jax.ShapeDtypeStruct((N, H), jnp.float32),
                   jax.ShapeDtypeStruct((N, H), jnp.float32)),
    )(x, Wd, Ws)

    ea = pl.pallas_call(
        _ea_body,
        grid=(E // EBLK,),
        in_specs=[
            pl.BlockSpec((EBLK, DE), lambda i: (i, 0)),
            pl.BlockSpec((DE, H), lambda i: (0, 0)),
            pl.BlockSpec((H,), lambda i: (0,)),
        ],
        out_specs=pl.BlockSpec((EBLK, H), lambda i: (i, 0)),
        out_shape=jax.ShapeDtypeStruct((E, H), jnp.float32),
    )(edge_attr, Wa, be1)

    hagg, cnt = _sc_edge(xd, xs, ea, dst, src)

    out = pl.pallas_call(
        _node_body,
        out_shape=jax.ShapeDtypeStruct((N, D), jnp.float32),
    )(hagg, cnt, x, We2, be2, Wn1[:D], Wn1[D:], bn1, Wn2, bn2)
    return out


# speedup vs baseline: 5.0745x; 1.6049x over previous
import functools

import jax
import jax.numpy as jnp
from jax import lax
from jax.experimental import pallas as pl
from jax.experimental.pallas import tpu as pltpu
from jax.experimental.pallas import tpu_sc as plsc

N = 10000
NPAD = 10240
E = 320000
D = 128
DE = 16
H = 128

NC = 2
NS = 16
NW = NC * NS
EPW = E // NW
B = 40
BP = 48
NBATCH = EPW // B
RPS = NPAD // NS


def _pre_body(x_ref, wd_ref, ws_ref, xd_ref, xs_ref):
    xv = x_ref[...]
    dn = (((1,), (0,)), ((), ()))
    xd_ref[...] = lax.dot_general(xv, wd_ref[...], dn,
                                  preferred_element_type=jnp.float32)
    xs_ref[...] = lax.dot_general(xv, ws_ref[...], dn,
                                  preferred_element_type=jnp.float32)


def _ea_body(ea_ref, wa_ref, b1_ref, out_ref):
    dn = (((1,), (0,)), ((), ()))
    out_ref[...] = lax.dot_general(ea_ref[...], wa_ref[...], dn,
                                   preferred_element_type=jnp.float32) + b1_ref[...]


def _node_body(hagg_ref, cts_ref, x_ref, w2_ref, b2_ref, wn1x_ref, wn1a_ref,
               bn1_ref, wn2_ref, bn2_ref, out_ref):
    dn = (((1,), (0,)), ((), ()))
    hsum = hagg_ref[0] + hagg_ref[1]
    aggp = lax.dot_general(hsum, w2_ref[...], dn,
                           preferred_element_type=jnp.float32)
    c = jnp.sum(cts_ref[...], axis=0)
    aggp = aggp + c[:, None] * b2_ref[...][None, :]
    agg = aggp[:N]
    pre = (lax.dot_general(x_ref[...], wn1x_ref[...], dn,
                           preferred_element_type=jnp.float32)
           + lax.dot_general(agg, wn1a_ref[...], dn,
                             preferred_element_type=jnp.float32)
           + bn1_ref[...])
    hn = jnp.maximum(pre, 0.0)
    out_ref[...] = lax.dot_general(hn, wn2_ref[...], dn,
                                   preferred_element_type=jnp.float32) + bn2_ref[...]


def _sc_edge_body(xd_hbm, xs_hbm, ea_hbm, dst_hbm, src_hbm,
                  out_hbm, cnt_hbm,
                  dsti0, srci0, xdv0, xsv0, hv0,
                  dsti1, srci1, xdv1, xsv1, hv1,
                  cntv, acc, dsem0, dsem1, gsem0, gsem1):
    cid = lax.axis_index("c")
    sid = lax.axis_index("s")
    wid = cid * NS + sid
    ebase = wid * EPW

    slots = ((dsti0, srci0, xdv0, xsv0, hv0, dsem0, gsem0),
             (dsti1, srci1, xdv1, xsv1, hv1, dsem1, gsem1))

    zero16 = jnp.zeros((16,), jnp.float32)
    one16 = jnp.full((16,), 1.0, jnp.float32)
    lane = lax.iota(jnp.int32, 16)
    tailmask = lane < (B - 2 * 16)

    def zrow(r, c):
        for j in range(H // 16):
            hv0[r, pl.ds(j * 16, 16)] = zero16
        return c
    lax.fori_loop(0, B, zrow, 0)
    for k in range(RPS // B):
        off = pl.multiple_of(sid * RPS + k * B, 8)
        pltpu.sync_copy(hv0, acc.at[pl.ds(off, B)])

    def crow(r, c):
        cntv[pl.ds(r * 16, 16)] = zero16
        return c
    lax.fori_loop(0, NPAD // 16, crow, 0)

    plsc.subcore_barrier()

    def idx_start(sl, b):
        d, s, _, _, _, dsem, _ = slots[sl]
        base = ebase + b * B
        pltpu.async_copy(dst_hbm.at[pl.ds(base, B)], d.at[pl.ds(0, B)], dsem)
        pltpu.async_copy(src_hbm.at[pl.ds(base, B)], s.at[pl.ds(0, B)], dsem)

    def idx_wait(sl, b):
        d, s, _, _, _, dsem, _ = slots[sl]
        base = ebase + b * B
        pltpu.make_async_copy(
            dst_hbm.at[pl.ds(base, B)], d.at[pl.ds(0, B)], dsem).wait()
        pltpu.make_async_copy(
            src_hbm.at[pl.ds(base, B)], s.at[pl.ds(0, B)], dsem).wait()

    def g_start(sl, b):
        d, s, xdv, xsv, hv, _, gsem = slots[sl]
        base = ebase + b * B
        pltpu.async_copy(xd_hbm.at[d.at[pl.ds(0, B)]], xdv, gsem)
        pltpu.async_copy(xs_hbm.at[s.at[pl.ds(0, B)]], xsv, gsem)
        pltpu.async_copy(ea_hbm.at[pl.ds(base, B)], hv, gsem)

    def g_wait(sl, b):
        d, s, xdv, xsv, hv, _, gsem = slots[sl]
        base = ebase + b * B
        pltpu.make_async_copy(xd_hbm.at[d.at[pl.ds(0, B)]], xdv, gsem).wait()
        pltpu.make_async_copy(xs_hbm.at[s.at[pl.ds(0, B)]], xsv, gsem).wait()
        pltpu.make_async_copy(ea_hbm.at[pl.ds(base, B)], hv, gsem).wait()

    idx_start(0, 0)
    idx_start(1, 1)
    idx_wait(0, 0)
    g_start(0, 0)

    def super_body(g, c):
        for j in range(2):
            b = g * 2 + j
            d, s, xdv, xsv, hv, dsem, gsem = slots[j]
            g_wait(j, b)

            @pl.when(b + 1 < NBATCH)
            def _():
                idx_wait(1 - j, b + 1)
                g_start(1 - j, b + 1)

            def row(r, cc):
                for jj in range(H // 16):
                    sl16 = pl.ds(jj * 16, 16)
                    t = hv[r, sl16] + xdv[r, sl16] + xsv[r, sl16]
                    hv[r, sl16] = jnp.maximum(t, 0.0)
                return cc
            lax.fori_loop(0, B, row, 0)

            for kk in range(B // 16):
                plsc.addupdate_scatter(cntv, [d[pl.ds(kk * 16, 16)]], one16)
            if B % 16:
                plsc.addupdate_scatter(
                    cntv, [d[pl.ds((B // 16) * 16, 16)]], one16, mask=tailmask)

            pltpu.sync_copy(hv, acc.at[d.at[pl.ds(0, B)]], add=True)

            @pl.when(b + 2 < NBATCH)
            def _():
                idx_start(j, b + 2)
        return c
    lax.fori_loop(0, NBATCH // 2, super_body, 0)

    pltpu.sync_copy(cntv, cnt_hbm.at[wid])

    plsc.subcore_barrier()
    off = pl.multiple_of(sid * RPS, 8)
    pltpu.sync_copy(acc.at[pl.ds(off, RPS)],
                    out_hbm.at[cid, pl.ds(off, RPS)])


_sc_edge = functools.partial(
    pl.kernel,
    mesh=plsc.VectorSubcoreMesh(core_axis_name="c", subcore_axis_name="s"),
    compiler_params=pltpu.CompilerParams(needs_layout_passes=False),
    out_type=(jax.ShapeDtypeStruct((NC, NPAD, H), jnp.float32),
              jax.ShapeDtypeStruct((NW, NPAD), jnp.float32)),
    scratch_types=[
        pltpu.VMEM((BP,), jnp.int32),
        pltpu.VMEM((BP,), jnp.int32),
        pltpu.VMEM((B, H), jnp.float32),
        pltpu.VMEM((B, H), jnp.float32),
        pltpu.VMEM((B, H), jnp.float32),
        pltpu.VMEM((BP,), jnp.int32),
        pltpu.VMEM((BP,), jnp.int32),
        pltpu.VMEM((B, H), jnp.float32),
        pltpu.VMEM((B, H), jnp.float32),
        pltpu.VMEM((B, H), jnp.float32),
        pltpu.VMEM((NPAD,), jnp.float32),
        pltpu.VMEM_SHARED((NPAD, H), jnp.float32),
        pltpu.SemaphoreType.DMA,
        pltpu.SemaphoreType.DMA,
        pltpu.SemaphoreType.DMA,
        pltpu.SemaphoreType.DMA,
    ],
)(_sc_edge_body)


EBLK = 8000


def kernel(x, edge_index, edge_attr, We1, be1, We2, be2, Wn1, bn1, Wn2, bn2):
    dst = edge_index[1].astype(jnp.int32)
    src = edge_index[0].astype(jnp.int32)
    Wa = We1[:DE]
    Wd = We1[DE:DE + D]
    Ws = We1[DE + D:]

    xd, xs = pl.pallas_call(
        _pre_body,
        out_shape=(jax.ShapeDtypeStruct((N, H), jnp.float32),
                   jax.ShapeDtypeStruct((N, H), jnp.float32)),
    )(x, Wd, Ws)

    ea = pl.pallas_call(
        _ea_body,
        grid=(E // EBLK,),
        in_specs=[
            pl.BlockSpec((EBLK, DE), lambda i: (i, 0)),
            pl.BlockSpec((DE, H), lambda i: (0, 0)),
            pl.BlockSpec((H,), lambda i: (0,)),
        ],
        out_specs=pl.BlockSpec((EBLK, H), lambda i: (i, 0)),
        out_shape=jax.ShapeDtypeStruct((E, H), jnp.float32),
    )(edge_attr, Wa, be1)

    hagg, cnt = _sc_edge(xd, xs, ea, dst, src)

    out = pl.pallas_call(
        _node_body,
        out_shape=jax.ShapeDtypeStruct((N, D), jnp.float32),
    )(hagg, cnt, x, We2, be2, Wn1[:D], Wn1[D:], bn1, Wn2, bn2)
    return out
